# baseline (speedup 1.0000x reference)
import jax
import jax.numpy as jnp
from jax.experimental import pallas as pl
from jax.experimental.pallas import tpu as pltpu

_BLOCK = 5000
_COLS = 2


def _linear_kernel(x_ref, w_ref, b_ref, o_ref):
    o_ref[...] = (
        jnp.dot(
            x_ref[...].astype(jnp.bfloat16),
            w_ref[...].astype(jnp.bfloat16),
            preferred_element_type=jnp.float32,
        )
        + b_ref[...]
    )


def kernel(x, A_indices, A_values, W, b):
    del A_indices, A_values
    n, d_in = x.shape
    d_out = W.shape[1]
    cb = d_out // _COLS
    return pl.pallas_call(
        _linear_kernel,
        grid=(pl.cdiv(n, _BLOCK), _COLS),
        compiler_params=pltpu.CompilerParams(
            dimension_semantics=("arbitrary", "arbitrary"),
        ),
        in_specs=[
            pl.BlockSpec((_BLOCK, d_in), lambda i, j: (i, 0)),
            pl.BlockSpec((d_in, cb), lambda i, j: (0, j)),
            pl.BlockSpec((1, cb), lambda i, j: (0, j)),
        ],
        out_specs=pl.BlockSpec((_BLOCK, cb), lambda i, j: (i, j)),
        out_shape=jax.ShapeDtypeStruct((n, d_out), x.dtype),
    )(x, W, b.reshape(1, d_out))


# speedup vs baseline: 1.4871x; 1.4871x over previous
import jax
import jax.numpy as jnp
from jax.experimental import pallas as pl
from jax.experimental.pallas import tpu as pltpu

_BLOCK = 5000


def _linear_kernel(x_ref, w_ref, b_ref, o_ref):
    o_ref[...] = (
        jnp.dot(x_ref[...], w_ref[...], preferred_element_type=jnp.float32)
        + b_ref[...]
    )


def kernel(x, A_indices, A_values, W, b):
    del A_indices, A_values
    n, d_in = x.shape
    d_out = W.shape[1]
    return pl.pallas_call(
        _linear_kernel,
        grid=(pl.cdiv(n, _BLOCK),),
        compiler_params=pltpu.CompilerParams(
            dimension_semantics=("arbitrary",),
        ),
        in_specs=[
            pl.BlockSpec((_BLOCK, d_in), lambda i: (i, 0)),
            pl.BlockSpec((d_in, d_out), lambda i: (0, 0)),
            pl.BlockSpec((1, d_out), lambda i: (0, 0)),
        ],
        out_specs=pl.BlockSpec((_BLOCK, d_out), lambda i: (i, 0)),
        out_shape=jax.ShapeDtypeStruct((n, d_out), x.dtype),
    )(x, W, b.reshape(1, d_out))
